# baseline (speedup 1.0000x reference)
import numpy as np
import jax
import jax.numpy as jnp
from jax.experimental import pallas as pl
from jax.experimental.pallas import tpu as pltpu

_H = 4
_HD = 32
_HID = 128
_THR = 0.7
_PREC = jax.lax.Precision.HIGHEST
_INTERPRET = False


def _dot(a, b, prec=_PREC):
    return jax.lax.dot_general(a, b, (((1,), (0,)), ((), ())),
                               precision=prec,
                               preferred_element_type=jnp.float32)


def _hi_lo(x):
    xi = jax.lax.bitcast_convert_type(x, jnp.int32)
    hi = jax.lax.bitcast_convert_type(xi & jnp.int32(-65536), jnp.float32)
    return hi, x - hi


def _pass1_body(xt_ref, xi_ref, lab_ref, wbt_ref, wbi_ref,
                btv_ref, biv_ref, blog_ref, ba_ref, u_ref, stats_ref):
    j = pl.program_id(0)
    bf16 = jnp.bfloat16
    xt_hi, xt_lo = _hi_lo(xt_ref[...])
    xi_hi, xi_lo = _hi_lo(xi_ref[...])
    xth = xt_hi.astype(bf16)
    xih = xi_hi.astype(bf16)
    xtc = jnp.concatenate([xth, xth, xt_lo.astype(bf16)], axis=1)
    xic = jnp.concatenate([xih, xih, xi_lo.astype(bf16)], axis=1)
    t_out = _dot(xtc, wbt_ref[...], prec=jax.lax.Precision.DEFAULT)
    i_out = _dot(xic, wbi_ref[...], prec=jax.lax.Precision.DEFAULT)
    t_v = t_out[:, 0:128] + btv_ref[...]
    i_v = i_out[:, 0:128] + biv_ref[...]
    log16 = t_out[:, 128:144] + i_out[:, 128:144] + blog_ref[...]
    a16 = t_out[:, 144:160] + i_out[:, 144:160] + ba_ref[...]
    s = 1.0 / (1.0 + jnp.exp(-log16))
    lane = jax.lax.broadcasted_iota(jnp.int32, (1, 16), 1)
    s_eff = jnp.where(lane >= 8, s * s, s)
    s_eff = jnp.where(lane >= 12, 1.0, s_eff)
    u_ref[...] = (s_eff * a16).T

    lab_row = lab_ref[0]
    gt = jnp.where(t_v > _THR, 1.0, 0.0)
    gi = jnp.where(i_v > _THR, 1.0, 0.0)
    g = jnp.concatenate([gt, gi, jnp.ones_like(gt)], axis=1)
    l2 = jnp.concatenate([jnp.ones_like(lab_row), lab_row], axis=0)
    r = jax.lax.dot_general(l2, g, (((1,), (0,)), ((), ())),
                            preferred_element_type=jnp.float32)

    @pl.when(j == 0)
    def _():
        stats_ref[...] = r

    @pl.when(j != 0)
    def _():
        stats_ref[...] += r


def _pass2_body(bf, stats_ref, u_ref, o_ref):
    st = stats_ref[...]
    n1 = st[0:1, 0:256]
    n11 = st[1:2, 0:256]
    nl128 = st[1:2, 256:384]
    nl = jnp.concatenate([nl128, nl128], axis=1)
    t = bf + 1e-6
    c11 = n11
    c10 = n1 - n11
    c01 = nl - n11
    c00 = bf - n1 - nl + n11
    ncol0 = bf - nl
    ncol1 = nl
    nrow0 = bf - n1
    nrow1 = n1
    e00 = ncol0 * nrow0 / t
    e01 = ncol0 * nrow1 / t
    e10 = ncol1 * nrow0 / t
    e11 = ncol1 * nrow1 / t
    chi = ((c00 - e00) ** 2 / (e00 + 1e-6) + (c01 - e01) ** 2 / (e01 + 1e-6)
           + (c10 - e10) ** 2 / (e10 + 1e-6) + (c11 - e11) ** 2 / (e11 + 1e-6))
    m = jnp.max(chi, axis=1, keepdims=True)
    alpha = chi / (m + 1e-6)

    rr = jax.lax.broadcasted_iota(jnp.int32, (256, 16), 0)
    qq = jax.lax.broadcasted_iota(jnp.int32, (256, 16), 1)
    ma = (jnp.where((rr < 4) & (qq == rr), 1.0, 0.0)
          + jnp.where((rr >= 128) & (rr < 132) & (qq == rr - 124), 1.0, 0.0))
    mb = jnp.where((rr < 4) & (qq == rr + 8), 1.0, 0.0)
    mc = jnp.where((rr >= 128) & (rr < 132) & (qq == rr - 120), 1.0, 0.0)
    p1 = _dot(alpha, ma)
    p2 = _dot(alpha, mb)
    p3 = _dot(alpha, mc)
    lane = jax.lax.broadcasted_iota(jnp.int32, (1, 16), 1)
    c16 = p1 - p2 * p3 + jnp.where(lane == 12, 1.0, 0.0)

    row = jax.lax.dot_general(
        c16, u_ref[...], (((1,), (0,)), ((), ())),
        precision=_PREC, preferred_element_type=jnp.float32)
    o_ref[...] = row[None]


def kernel(text_vec, image_vec, label, Wtq, btq, Wtk, btk, Wtv, btv,
           Wiq, biq, Wik, bik, Wiv, biv, attn_W, attn_b, Wout, bout):
    B = text_vec.shape[0]
    f32 = jnp.float32

    blockmap = np.repeat(np.eye(_H, dtype=np.float32), _HD, axis=0)
    z4 = np.zeros((_HID, _H), np.float32)
    rep3 = np.concatenate([blockmap, blockmap, blockmap, z4], axis=1)
    g0 = np.concatenate([blockmap, z4, z4, z4], axis=1)
    g12 = np.concatenate([z4, blockmap, blockmap, z4], axis=1)
    e12 = np.zeros((16,), np.float32)
    e12[12] = 1.0

    aq = attn_W[:, :_HD].reshape(-1)
    ak = attn_W[:, _HD:].reshape(-1)
    wf = Wout[0, 0:_HID]
    wqt = Wout[0, _HID:2 * _HID]
    wqi = Wout[0, 2 * _HID:3 * _HID]

    mlog_q = aq[:, None] * rep3
    mlog_k = ak[:, None] * rep3
    ma_t = wf[:, None] * g0
    ma_i = wf[:, None] * g12
    mp_t = wqt[:, None] * e12[None, :]
    mp_i = wqi[:, None] * e12[None, :]

    hp = jax.lax.Precision.HIGHEST
    mm = lambda a, b: jnp.matmul(a, b, precision=hp)
    wTf = jnp.concatenate(
        [mm(Wtq.T, mlog_q), mm(Wtv.T, ma_t) + mm(Wtq.T, mp_t)], axis=1)
    wIf = jnp.concatenate(
        [mm(Wik.T, mlog_k), mm(Wiv.T, ma_i) + mm(Wiq.T, mp_i)], axis=1)

    def _hi_lo_w(w):
        wi = jax.lax.bitcast_convert_type(w, jnp.int32)
        hi = jax.lax.bitcast_convert_type(wi & jnp.int32(-65536), f32)
        return hi, w - hi

    z32 = jnp.zeros((256, 32), f32)

    def _wbig(wv, wfunc):
        hi, lo = _hi_lo_w(wv)
        return jnp.concatenate([
            jnp.concatenate([hi, wfunc], axis=1),
            jnp.concatenate([lo, z32], axis=1),
            jnp.concatenate([hi, z32], axis=1),
        ], axis=0).astype(jnp.bfloat16)

    wBT = _wbig(Wtv.T, wTf)
    wBI = _wbig(Wiv.T, wIf)
    attn_b_rep = jnp.concatenate([attn_b, attn_b, attn_b, jnp.zeros(4, f32)])
    blog = (mm(btq[None, :], mlog_q) + mm(bik[None, :], mlog_k)
            + attn_b_rep[None, :])
    bA = (mm(btv[None, :], ma_t) + mm(btq[None, :], mp_t)
          + mm(biv[None, :], ma_i) + mm(biq[None, :], mp_i)
          + bout[0] * e12[None, :])
    btv_r = btv[None, :]
    biv_r = biv[None, :]

    bB = 4096
    while B % bB != 0:
        bB //= 2
    nb = B // bB
    labf = label.astype(f32).reshape(nb, 1, bB)
    row = lambda j: (j, 0)
    fixed2 = lambda j: (0, 0)
    u, stats = pl.pallas_call(
        _pass1_body,
        grid=(nb,),
        in_specs=[
            pl.BlockSpec((bB, 256), row),
            pl.BlockSpec((bB, 256), row),
            pl.BlockSpec((1, 1, bB), lambda j: (j, 0, 0)),
            pl.BlockSpec((768, 160), fixed2),
            pl.BlockSpec((768, 160), fixed2),
            pl.BlockSpec((1, 128), fixed2),
            pl.BlockSpec((1, 128), fixed2),
            pl.BlockSpec((1, 16), fixed2),
            pl.BlockSpec((1, 16), fixed2),
        ],
        out_specs=[
            pl.BlockSpec((16, bB), lambda j: (0, j)),
            pl.BlockSpec((2, 384), fixed2),
        ],
        out_shape=[
            jax.ShapeDtypeStruct((16, B), f32),
            jax.ShapeDtypeStruct((2, 384), f32),
        ],
        compiler_params=pltpu.CompilerParams(
            dimension_semantics=(pltpu.ARBITRARY,),
            vmem_limit_bytes=50 * 1024 * 1024,
        ),
        name="han_pass1",
        interpret=_INTERPRET,
    )(text_vec, image_vec, labf, wBT, wBI, btv_r, biv_r, blog, bA)

    bB2 = 8192
    while B % bB2 != 0:
        bB2 //= 2
    nb2 = B // bB2
    out = pl.pallas_call(
        lambda *a: _pass2_body(jnp.float32(B), *a),
        grid=(nb2,),
        in_specs=[
            pl.BlockSpec((2, 384), fixed2),
            pl.BlockSpec((16, bB2), lambda j: (0, j)),
        ],
        out_specs=pl.BlockSpec((1, 1, bB2), lambda j: (j, 0, 0)),
        out_shape=jax.ShapeDtypeStruct((nb2, 1, bB2), f32),
        compiler_params=pltpu.CompilerParams(
            dimension_semantics=(pltpu.ARBITRARY,),
            vmem_limit_bytes=50 * 1024 * 1024,
        ),
        name="han_pass2",
        interpret=_INTERPRET,
    )(stats, u)
    return out.reshape(B, 1)


# speedup vs baseline: 1.0810x; 1.0810x over previous
import numpy as np
import jax
import jax.numpy as jnp
from jax.experimental import pallas as pl
from jax.experimental.pallas import tpu as pltpu

_H = 4
_HD = 32
_HID = 128
_THR = 0.7
_PREC = jax.lax.Precision.HIGHEST
_INTERPRET = False


def _dot(a, b, prec=_PREC):
    return jax.lax.dot_general(a, b, (((1,), (0,)), ((), ())),
                               precision=prec,
                               preferred_element_type=jnp.float32)


def _hi_lo(x):
    xi = jax.lax.bitcast_convert_type(x, jnp.int32)
    hi = jax.lax.bitcast_convert_type(xi & jnp.int32(-65536), jnp.float32)
    return hi, x - hi


def _pass1_body(xt_ref, xi_ref, lab_ref, wbt_ref, wbi_ref,
                btv_ref, biv_ref, blog_ref, ba_ref, u_ref, stats_ref):
    j = pl.program_id(0)
    xt_hi, xt_lo = _hi_lo(xt_ref[...])
    xi_hi, xi_lo = _hi_lo(xi_ref[...])
    xtc = jnp.concatenate([xt_hi, xt_hi, xt_lo], axis=1)
    xic = jnp.concatenate([xi_hi, xi_hi, xi_lo], axis=1)
    t_out = _dot(xtc, wbt_ref[...], prec=jax.lax.Precision.DEFAULT)
    i_out = _dot(xic, wbi_ref[...], prec=jax.lax.Precision.DEFAULT)
    t_v = t_out[:, 0:128] + btv_ref[...]
    i_v = i_out[:, 0:128] + biv_ref[...]
    log16 = t_out[:, 128:144] + i_out[:, 128:144] + blog_ref[...]
    a16 = t_out[:, 144:160] + i_out[:, 144:160] + ba_ref[...]
    s = 1.0 / (1.0 + jnp.exp(-log16))
    lane = jax.lax.broadcasted_iota(jnp.int32, (1, 16), 1)
    s_eff = jnp.where(lane >= 8, s * s, s)
    s_eff = jnp.where(lane >= 12, 1.0, s_eff)
    u_ref[...] = (s_eff * a16).T

    lab_row = lab_ref[0]
    gt = jnp.where(t_v > _THR, 1.0, 0.0)
    gi = jnp.where(i_v > _THR, 1.0, 0.0)
    g = jnp.concatenate([gt, gi, jnp.ones_like(gt)], axis=1)
    l2 = jnp.concatenate([jnp.ones_like(lab_row), lab_row], axis=0)
    r = jax.lax.dot_general(l2, g, (((1,), (0,)), ((), ())),
                            preferred_element_type=jnp.float32)

    @pl.when(j == 0)
    def _():
        stats_ref[...] = r

    @pl.when(j != 0)
    def _():
        stats_ref[...] += r


def _pass2_body(bf, stats_ref, u_ref, o_ref):
    st = stats_ref[...]
    n1 = st[0:1, 0:256]
    n11 = st[1:2, 0:256]
    nl128 = st[1:2, 256:384]
    nl = jnp.concatenate([nl128, nl128], axis=1)
    t = bf + 1e-6
    c11 = n11
    c10 = n1 - n11
    c01 = nl - n11
    c00 = bf - n1 - nl + n11
    ncol0 = bf - nl
    ncol1 = nl
    nrow0 = bf - n1
    nrow1 = n1
    e00 = ncol0 * nrow0 / t
    e01 = ncol0 * nrow1 / t
    e10 = ncol1 * nrow0 / t
    e11 = ncol1 * nrow1 / t
    chi = ((c00 - e00) ** 2 / (e00 + 1e-6) + (c01 - e01) ** 2 / (e01 + 1e-6)
           + (c10 - e10) ** 2 / (e10 + 1e-6) + (c11 - e11) ** 2 / (e11 + 1e-6))
    m = jnp.max(chi, axis=1, keepdims=True)
    alpha = chi / (m + 1e-6)

    rr = jax.lax.broadcasted_iota(jnp.int32, (256, 16), 0)
    qq = jax.lax.broadcasted_iota(jnp.int32, (256, 16), 1)
    ma = (jnp.where((rr < 4) & (qq == rr), 1.0, 0.0)
          + jnp.where((rr >= 128) & (rr < 132) & (qq == rr - 124), 1.0, 0.0))
    mb = jnp.where((rr < 4) & (qq == rr + 8), 1.0, 0.0)
    mc = jnp.where((rr >= 128) & (rr < 132) & (qq == rr - 120), 1.0, 0.0)
    p1 = _dot(alpha, ma)
    p2 = _dot(alpha, mb)
    p3 = _dot(alpha, mc)
    lane = jax.lax.broadcasted_iota(jnp.int32, (1, 16), 1)
    c16 = p1 - p2 * p3 + jnp.where(lane == 12, 1.0, 0.0)

    row = jax.lax.dot_general(
        c16, u_ref[...], (((1,), (0,)), ((), ())),
        precision=_PREC, preferred_element_type=jnp.float32)
    o_ref[...] = row[None]


def kernel(text_vec, image_vec, label, Wtq, btq, Wtk, btk, Wtv, btv,
           Wiq, biq, Wik, bik, Wiv, biv, attn_W, attn_b, Wout, bout):
    B = text_vec.shape[0]
    f32 = jnp.float32

    blockmap = np.repeat(np.eye(_H, dtype=np.float32), _HD, axis=0)
    z4 = np.zeros((_HID, _H), np.float32)
    rep3 = np.concatenate([blockmap, blockmap, blockmap, z4], axis=1)
    g0 = np.concatenate([blockmap, z4, z4, z4], axis=1)
    g12 = np.concatenate([z4, blockmap, blockmap, z4], axis=1)
    e12 = np.zeros((16,), np.float32)
    e12[12] = 1.0

    aq = attn_W[:, :_HD].reshape(-1)
    ak = attn_W[:, _HD:].reshape(-1)
    wf = Wout[0, 0:_HID]
    wqt = Wout[0, _HID:2 * _HID]
    wqi = Wout[0, 2 * _HID:3 * _HID]

    mlog_q = aq[:, None] * rep3
    mlog_k = ak[:, None] * rep3
    ma_t = wf[:, None] * g0
    ma_i = wf[:, None] * g12
    mp_t = wqt[:, None] * e12[None, :]
    mp_i = wqi[:, None] * e12[None, :]

    hp = jax.lax.Precision.HIGHEST
    mm = lambda a, b: jnp.matmul(a, b, precision=hp)
    wTf = jnp.concatenate(
        [mm(Wtq.T, mlog_q), mm(Wtv.T, ma_t) + mm(Wtq.T, mp_t)], axis=1)
    wIf = jnp.concatenate(
        [mm(Wik.T, mlog_k), mm(Wiv.T, ma_i) + mm(Wiq.T, mp_i)], axis=1)

    def _hi_lo_w(w):
        wi = jax.lax.bitcast_convert_type(w, jnp.int32)
        hi = jax.lax.bitcast_convert_type(wi & jnp.int32(-65536), f32)
        return hi, w - hi

    z32 = jnp.zeros((256, 32), f32)

    def _wbig(wv, wfunc):
        hi, lo = _hi_lo_w(wv)
        return jnp.concatenate([
            jnp.concatenate([hi, wfunc], axis=1),
            jnp.concatenate([lo, z32], axis=1),
            jnp.concatenate([hi, z32], axis=1),
        ], axis=0)

    wBT = _wbig(Wtv.T, wTf)
    wBI = _wbig(Wiv.T, wIf)
    attn_b_rep = jnp.concatenate([attn_b, attn_b, attn_b, jnp.zeros(4, f32)])
    blog = (mm(btq[None, :], mlog_q) + mm(bik[None, :], mlog_k)
            + attn_b_rep[None, :])
    bA = (mm(btv[None, :], ma_t) + mm(btq[None, :], mp_t)
          + mm(biv[None, :], ma_i) + mm(biq[None, :], mp_i)
          + bout[0] * e12[None, :])
    btv_r = btv[None, :]
    biv_r = biv[None, :]

    bB = 4096
    while B % bB != 0:
        bB //= 2
    nb = B // bB
    labf = label.astype(f32).reshape(nb, 1, bB)
    row = lambda j: (j, 0)
    fixed2 = lambda j: (0, 0)
    u, stats = pl.pallas_call(
        _pass1_body,
        grid=(nb,),
        in_specs=[
            pl.BlockSpec((bB, 256), row),
            pl.BlockSpec((bB, 256), row),
            pl.BlockSpec((1, 1, bB), lambda j: (j, 0, 0)),
            pl.BlockSpec((768, 160), fixed2),
            pl.BlockSpec((768, 160), fixed2),
            pl.BlockSpec((1, 128), fixed2),
            pl.BlockSpec((1, 128), fixed2),
            pl.BlockSpec((1, 16), fixed2),
            pl.BlockSpec((1, 16), fixed2),
        ],
        out_specs=[
            pl.BlockSpec((16, bB), lambda j: (0, j)),
            pl.BlockSpec((2, 384), fixed2),
        ],
        out_shape=[
            jax.ShapeDtypeStruct((16, B), f32),
            jax.ShapeDtypeStruct((2, 384), f32),
        ],
        compiler_params=pltpu.CompilerParams(
            dimension_semantics=(pltpu.ARBITRARY,),
            vmem_limit_bytes=50 * 1024 * 1024,
        ),
        name="han_pass1",
        interpret=_INTERPRET,
    )(text_vec, image_vec, labf, wBT, wBI, btv_r, biv_r, blog, bA)

    bB2 = 16384
    while B % bB2 != 0:
        bB2 //= 2
    nb2 = B // bB2
    out = pl.pallas_call(
        lambda *a: _pass2_body(jnp.float32(B), *a),
        grid=(nb2,),
        in_specs=[
            pl.BlockSpec((2, 384), fixed2),
            pl.BlockSpec((16, bB2), lambda j: (0, j)),
        ],
        out_specs=pl.BlockSpec((1, 1, bB2), lambda j: (j, 0, 0)),
        out_shape=jax.ShapeDtypeStruct((nb2, 1, bB2), f32),
        compiler_params=pltpu.CompilerParams(
            dimension_semantics=(pltpu.ARBITRARY,),
            vmem_limit_bytes=50 * 1024 * 1024,
        ),
        name="han_pass2",
        interpret=_INTERPRET,
    )(stats, u)
    return out.reshape(B, 1)
